# baseline (speedup 1.0000x reference)
import functools

import jax
import jax.numpy as jnp
import numpy as np
from jax.experimental import pallas as pl
from jax.experimental.pallas import tpu as pltpu

B, S, QH, KVH = 1, 2048, 12, 4
D, DV = 64, 64
KS, STRIDE, SEL, TOPN, WIN = 32, 16, 64, 16, 512
SCALE = D ** -0.5
NC = (S - KS) // STRIDE + 1
NCP = 128
NB = S // SEL
G = QH // KVH
NCHUNK = S // STRIDE
QT = 512
QTS = 256

_NEG_INF = float("-inf")


def _compress_kernel(kr_ref, vr_ref, w1k_ref, w2k_ref, w1v_ref, w2v_ref,
                     ck_ref, cv_ref):
    w1k, w2k = w1k_ref[...], w2k_ref[...]
    w1v, w2v = w1v_ref[...], w2v_ref[...]
    for h in range(KVH):
        ck = kr_ref[h]
        ckr = jnp.roll(ck, -1, axis=0)
        ck_ref[h] = (jnp.dot(ck, w1k, preferred_element_type=jnp.float32)
                     + jnp.dot(ckr, w2k, preferred_element_type=jnp.float32))
        cv = vr_ref[h]
        cvr = jnp.roll(cv, -1, axis=0)
        cv_ref[h] = (jnp.dot(cv, w1v, preferred_element_type=jnp.float32)
                     + jnp.dot(cvr, w2v, preferred_element_type=jnp.float32))


def _masked_softmax(s, mask):
    s = jnp.where(mask, s, _NEG_INF)
    m = jnp.max(s, axis=-1, keepdims=True)
    m = jnp.where(jnp.isfinite(m), m, 0.0)
    e = jnp.exp(s - m)
    return e / jnp.maximum(jnp.sum(e, axis=-1, keepdims=True), 1e-30)


def _cmp_attn_kernel(qt_ref, ck_ref, cv_ref, m_ref, ri_ref, rj_ref,
                     cmp_o_ref, selm_ref):
    i = pl.program_id(1)
    base = i * QT
    qb = qt_ref[0].reshape(G * QT, D)
    ck = ck_ref[0]
    cv = cv_ref[0]
    s = jax.lax.dot_general(qb, ck, (((1,), (1,)), ((), ())),
                            preferred_element_type=jnp.float32) * SCALE
    row = jax.lax.broadcasted_iota(jnp.int32, (G * QT, NCP), 0)
    col = jax.lax.broadcasted_iota(jnp.int32, (G * QT, NCP), 1)
    t = base + row % QT
    mask = t >= (col * STRIDE + KS - 1)
    p = _masked_softmax(s, mask)
    cmp_o_ref[0] = jax.lax.dot_general(
        p, cv, (((1,), (0,)), ((), ())),
        preferred_element_type=jnp.float32).reshape(G, QT, DV)

    @pl.when(base >= S // 2)
    def _selmask():
        pj = jax.lax.dot_general(p, m_ref[...], (((1,), (0,)), ((), ())),
                                 preferred_element_type=jnp.float32)
        pj = pj.reshape(G, QT, NB).sum(axis=0)
        trow = base + jax.lax.broadcasted_iota(jnp.int32, (QT, NB), 0)
        jcol = jax.lax.broadcasted_iota(jnp.int32, (QT, NB), 1)
        cur = trow // SEL
        force = (jcol == cur).astype(jnp.float32) + (jcol == 0).astype(jnp.float32)
        sc = pj + 1e9 * force
        xi = jax.lax.dot_general(sc, ri_ref[...], (((1,), (0,)), ((), ())),
                                 preferred_element_type=jnp.float32)
        xj = jax.lax.dot_general(sc, rj_ref[...], (((1,), (0,)), ((), ())),
                                 preferred_element_type=jnp.float32)
        f = jax.lax.broadcasted_iota(jnp.int32, (QT, NB * NB), 1)
        ii = f % NB
        jj = f // NB
        beats = ((xi > xj) | ((xi == xj) & (ii < jj))).astype(jnp.float32)
        rank = jax.lax.dot_general(beats, rj_ref[...], (((1,), (1,)), ((), ())),
                                   preferred_element_type=jnp.float32)
        selm = (rank < TOPN).astype(jnp.float32)
        selm_ref[0] = jnp.concatenate(
            [selm, jnp.zeros((QT, 128 - NB), jnp.float32)], axis=1)


def _selwin_kernel(qt_ref, kt_ref, vt_ref, selm_ref, cmp_o_ref, e_ref, wg_ref,
                   bg_ref, out_ref, *, qbase, kl, wlo, need_sel):
    rows = G * QTS
    qb = qt_ref[0].reshape(rows, D)
    kh = kt_ref[0]
    vh = vt_ref[0]
    s = jax.lax.dot_general(qb, kh, (((1,), (1,)), ((), ())),
                            preferred_element_type=jnp.float32) * SCALE
    row = jax.lax.broadcasted_iota(jnp.int32, (rows, kl), 0)
    tk = jax.lax.broadcasted_iota(jnp.int32, (rows, kl), 1)
    tq = qbase + row % QTS
    causal = tq >= tk

    if need_sel:
        posf = jax.lax.dot_general(selm_ref[0][:, :NB], e_ref[...],
                                   (((1,), (0,)), ((), ())),
                                   preferred_element_type=jnp.float32)
        pos = jnp.concatenate([posf] * G, axis=0) > 0.5
        sel_m = pos & causal
    else:
        sel_m = causal
    p_sel = _masked_softmax(s, sel_m)
    o_sel = jax.lax.dot_general(p_sel, vh, (((1,), (0,)), ((), ())),
                                preferred_element_type=jnp.float32)

    single = (not need_sel) and wlo == 0 and qbase + QTS <= WIN + 1
    if single:
        o_win = o_sel
    else:
        sw = s[:, wlo:kl]
        win_m = causal[:, wlo:kl] & (tk[:, wlo:kl] >= tq[:, :1] - WIN)
        p_win = _masked_softmax(sw, win_m)
        o_win = jax.lax.dot_general(p_win, vh[wlo:kl], (((1,), (0,)), ((), ())),
                                    preferred_element_type=jnp.float32)

    z = jax.lax.dot_general(qb, wg_ref[...], (((1,), (1,)), ((), ())),
                            preferred_element_type=jnp.float32) + bg_ref[...]
    gate = jax.nn.sigmoid(z)
    cmp_rows = cmp_o_ref[0].reshape(rows, DV)
    out = (gate[:, 0:1] * cmp_rows + gate[:, 1:2] * o_sel + gate[:, 2:3] * o_win)
    out_ref[0] = out.reshape(G, QTS, DV)


def _expand_matrix():
    e = (np.arange(S)[None, :] // SEL == np.arange(NB)[:, None]).astype(np.float32)
    return jnp.asarray(e)


def _overlap_matrix():
    cmp_start = np.arange(NCP) * STRIDE
    sel_start = np.arange(NB) * SEL
    ov = ((cmp_start[:, None] < sel_start[None, :] + SEL)
          & (cmp_start[:, None] + KS > sel_start[None, :])).astype(np.float32)
    ov[NC:] = 0.0
    return jnp.asarray(ov)


def _rank_matrices():
    f = np.arange(NB * NB)
    ri = (f[None, :] % NB == np.arange(NB)[:, None]).astype(np.float32)
    rj = (f[None, :] // NB == np.arange(NB)[:, None]).astype(np.float32)
    return jnp.asarray(ri), jnp.asarray(rj)


@jax.jit
def kernel(q, k, v, w_kc, w_vc, w_gate, b_gate):
    qs = q[0]
    ks = k[0]
    vs = v[0]
    kr = ks.transpose(1, 0, 2).reshape(KVH, NCHUNK, STRIDE * D)
    vr = vs.transpose(1, 0, 2).reshape(KVH, NCHUNK, STRIDE * DV)
    w1k, w2k = w_kc[: STRIDE * D], w_kc[STRIDE * D:]
    w1v, w2v = w_vc[: STRIDE * DV], w_vc[STRIDE * DV:]

    cmp_k, cmp_v = pl.pallas_call(
        _compress_kernel,
        out_shape=(
            jax.ShapeDtypeStruct((KVH, NCP, D), jnp.float32),
            jax.ShapeDtypeStruct((KVH, NCP, DV), jnp.float32),
        ),
    )(kr, vr, w1k, w2k, w1v, w2v)

    qg = qs.reshape(S, KVH, G, D).transpose(1, 2, 0, 3)
    m = _overlap_matrix()
    ri, rj = _rank_matrices()
    cmp_o, selm = pl.pallas_call(
        _cmp_attn_kernel,
        grid=(KVH, S // QT),
        in_specs=[
            pl.BlockSpec((1, G, QT, D), lambda h, i: (h, 0, i, 0)),
            pl.BlockSpec((1, NCP, D), lambda h, i: (h, 0, 0)),
            pl.BlockSpec((1, NCP, DV), lambda h, i: (h, 0, 0)),
            pl.BlockSpec((NCP, NB), lambda h, i: (0, 0)),
            pl.BlockSpec((NB, NB * NB), lambda h, i: (0, 0)),
            pl.BlockSpec((NB, NB * NB), lambda h, i: (0, 0)),
        ],
        out_specs=(
            pl.BlockSpec((1, G, QT, DV), lambda h, i: (h, 0, i, 0)),
            pl.BlockSpec((1, QT, 128), lambda h, i: (h, i, 0)),
        ),
        out_shape=(
            jax.ShapeDtypeStruct((KVH, G, S, DV), jnp.float32),
            jax.ShapeDtypeStruct((KVH, S, 128), jnp.float32),
        ),
    )(qg, cmp_k, cmp_v, m, ri, rj)

    kt = ks.transpose(1, 0, 2)
    vt = vs.transpose(1, 0, 2)
    e = _expand_matrix()
    wg = jnp.zeros((8, D), jnp.float32).at[:3].set(w_gate)
    bg = jnp.zeros((1, 8), jnp.float32).at[0, :3].set(b_gate)

    outs = []
    for c in range(S // QTS):
        qbase = c * QTS
        kl = qbase + QTS
        wlo = max(0, qbase - WIN)
        need_sel = qbase >= S // 2
        outs.append(pl.pallas_call(
            functools.partial(_selwin_kernel, qbase=qbase, kl=kl, wlo=wlo,
                              need_sel=need_sel),
            grid=(KVH,),
            in_specs=[
                pl.BlockSpec((1, G, QTS, D), lambda h, cc=c: (h, 0, cc, 0)),
                pl.BlockSpec((1, kl, D), lambda h: (h, 0, 0)),
                pl.BlockSpec((1, kl, DV), lambda h: (h, 0, 0)),
                pl.BlockSpec((1, QTS, 128), lambda h, cc=c: (h, cc, 0)),
                pl.BlockSpec((1, G, QTS, DV), lambda h, cc=c: (h, 0, cc, 0)),
                pl.BlockSpec((NB, kl), lambda h: (0, 0)),
                pl.BlockSpec((8, D), lambda h: (0, 0)),
                pl.BlockSpec((1, 8), lambda h: (0, 0)),
            ],
            out_specs=pl.BlockSpec((1, G, QTS, DV), lambda h: (h, 0, 0, 0)),
            out_shape=jax.ShapeDtypeStruct((KVH, G, QTS, DV), jnp.float32),
        )(qg, kt[:, :kl], vt[:, :kl], selm, cmp_o, e[:, :kl], wg, bg))

    out = jnp.concatenate(outs, axis=2)
    return out.transpose(2, 0, 1, 3).reshape(1, S, QH, DV)


# speedup vs baseline: 1.1475x; 1.1475x over previous
import functools

import jax
import jax.numpy as jnp
import numpy as np
from jax.experimental import pallas as pl
from jax.experimental.pallas import tpu as pltpu

B, S, QH, KVH = 1, 2048, 12, 4
D, DV = 64, 64
KS, STRIDE, SEL, TOPN, WIN = 32, 16, 64, 16, 512
SCALE = D ** -0.5
NC = (S - KS) // STRIDE + 1
NCP = 128
NB = S // SEL
G = QH // KVH
NCHUNK = S // STRIDE
QT = 512
QTS = 512

_NEG_INF = float("-inf")


def _compress_kernel(kr_ref, vr_ref, w1k_ref, w2k_ref, w1v_ref, w2v_ref,
                     ck_ref, cv_ref):
    w1k, w2k = w1k_ref[...], w2k_ref[...]
    w1v, w2v = w1v_ref[...], w2v_ref[...]
    for h in range(KVH):
        ck = kr_ref[h]
        ckr = jnp.roll(ck, -1, axis=0)
        ck_ref[h] = (jnp.dot(ck, w1k, preferred_element_type=jnp.float32)
                     + jnp.dot(ckr, w2k, preferred_element_type=jnp.float32))
        cv = vr_ref[h]
        cvr = jnp.roll(cv, -1, axis=0)
        cv_ref[h] = (jnp.dot(cv, w1v, preferred_element_type=jnp.float32)
                     + jnp.dot(cvr, w2v, preferred_element_type=jnp.float32))


def _masked_softmax(s, mask):
    s = jnp.where(mask, s, _NEG_INF)
    m = jnp.max(s, axis=-1, keepdims=True)
    m = jnp.where(jnp.isfinite(m), m, 0.0)
    e = jnp.exp(s - m)
    return e / jnp.maximum(jnp.sum(e, axis=-1, keepdims=True), 1e-30)


def _cmp_attn_kernel(qt_ref, ck_ref, cv_ref, m_ref, ri_ref, rj_ref,
                     cmp_o_ref, selm_ref):
    i = pl.program_id(1)
    base = i * QT
    qb = qt_ref[0].reshape(G * QT, D)
    ck = ck_ref[0]
    cv = cv_ref[0]
    s = jax.lax.dot_general(qb, ck, (((1,), (1,)), ((), ())),
                            preferred_element_type=jnp.float32) * SCALE
    row = jax.lax.broadcasted_iota(jnp.int32, (G * QT, NCP), 0)
    col = jax.lax.broadcasted_iota(jnp.int32, (G * QT, NCP), 1)
    t = base + row % QT
    mask = t >= (col * STRIDE + KS - 1)
    p = _masked_softmax(s, mask)
    cmp_o_ref[0] = jax.lax.dot_general(
        p, cv, (((1,), (0,)), ((), ())),
        preferred_element_type=jnp.float32).reshape(G, QT, DV)

    @pl.when(base >= S // 2)
    def _selmask():
        pj = jax.lax.dot_general(p, m_ref[...], (((1,), (0,)), ((), ())),
                                 preferred_element_type=jnp.float32)
        pj = pj.reshape(G, QT, NB).sum(axis=0)
        trow = base + jax.lax.broadcasted_iota(jnp.int32, (QT, NB), 0)
        jcol = jax.lax.broadcasted_iota(jnp.int32, (QT, NB), 1)
        cur = trow // SEL
        force = (jcol == cur).astype(jnp.float32) + (jcol == 0).astype(jnp.float32)
        sc = pj + 1e9 * force
        xi = jax.lax.dot_general(sc, ri_ref[...], (((1,), (0,)), ((), ())),
                                 preferred_element_type=jnp.float32)
        xj = jax.lax.dot_general(sc, rj_ref[...], (((1,), (0,)), ((), ())),
                                 preferred_element_type=jnp.float32)
        f = jax.lax.broadcasted_iota(jnp.int32, (QT, NB * NB), 1)
        ii = f % NB
        jj = f // NB
        beats = ((xi > xj) | ((xi == xj) & (ii < jj))).astype(jnp.float32)
        rank = jax.lax.dot_general(beats, rj_ref[...], (((1,), (1,)), ((), ())),
                                   preferred_element_type=jnp.float32)
        selm = (rank < TOPN).astype(jnp.float32)
        selm_ref[0] = jnp.concatenate(
            [selm, jnp.zeros((QT, 128 - NB), jnp.float32)], axis=1)


def _selwin_kernel(qt_ref, kt_ref, vt_ref, selm_ref, cmp_o_ref, e_ref, wg_ref,
                   bg_ref, out_ref, *, qbase, kl, wlo, need_sel):
    rows = G * QTS
    qb = qt_ref[0].reshape(rows, D)
    kh = kt_ref[0]
    vh = vt_ref[0]
    s = jax.lax.dot_general(qb, kh, (((1,), (1,)), ((), ())),
                            preferred_element_type=jnp.float32) * SCALE
    row = jax.lax.broadcasted_iota(jnp.int32, (rows, kl), 0)
    tk = jax.lax.broadcasted_iota(jnp.int32, (rows, kl), 1)
    tq = qbase + row % QTS
    causal = tq >= tk

    if need_sel:
        posf = jax.lax.dot_general(selm_ref[0][:, :NB], e_ref[...],
                                   (((1,), (0,)), ((), ())),
                                   preferred_element_type=jnp.float32)
        pos = jnp.concatenate([posf] * G, axis=0) > 0.5
        sel_m = pos & causal
    else:
        sel_m = causal
    p_sel = _masked_softmax(s, sel_m)
    o_sel = jax.lax.dot_general(p_sel, vh, (((1,), (0,)), ((), ())),
                                preferred_element_type=jnp.float32)

    single = (not need_sel) and wlo == 0 and qbase + QTS <= WIN + 1
    if single:
        o_win = o_sel
    else:
        sw = s[:, wlo:kl]
        win_m = causal[:, wlo:kl] & (tk[:, wlo:kl] >= tq[:, :1] - WIN)
        p_win = _masked_softmax(sw, win_m)
        o_win = jax.lax.dot_general(p_win, vh[wlo:kl], (((1,), (0,)), ((), ())),
                                    preferred_element_type=jnp.float32)

    z = jax.lax.dot_general(qb, wg_ref[...], (((1,), (1,)), ((), ())),
                            preferred_element_type=jnp.float32) + bg_ref[...]
    gate = jax.nn.sigmoid(z)
    cmp_rows = cmp_o_ref[0].reshape(rows, DV)
    out = (gate[:, 0:1] * cmp_rows + gate[:, 1:2] * o_sel + gate[:, 2:3] * o_win)
    out_ref[0] = out.reshape(G, QTS, DV)


def _expand_matrix():
    e = (np.arange(S)[None, :] // SEL == np.arange(NB)[:, None]).astype(np.float32)
    return jnp.asarray(e)


def _overlap_matrix():
    cmp_start = np.arange(NCP) * STRIDE
    sel_start = np.arange(NB) * SEL
    ov = ((cmp_start[:, None] < sel_start[None, :] + SEL)
          & (cmp_start[:, None] + KS > sel_start[None, :])).astype(np.float32)
    ov[NC:] = 0.0
    return jnp.asarray(ov)


def _rank_matrices():
    f = np.arange(NB * NB)
    ri = (f[None, :] % NB == np.arange(NB)[:, None]).astype(np.float32)
    rj = (f[None, :] // NB == np.arange(NB)[:, None]).astype(np.float32)
    return jnp.asarray(ri), jnp.asarray(rj)


@jax.jit
def kernel(q, k, v, w_kc, w_vc, w_gate, b_gate):
    qs = q[0]
    ks = k[0]
    vs = v[0]
    kr = ks.transpose(1, 0, 2).reshape(KVH, NCHUNK, STRIDE * D)
    vr = vs.transpose(1, 0, 2).reshape(KVH, NCHUNK, STRIDE * DV)
    w1k, w2k = w_kc[: STRIDE * D], w_kc[STRIDE * D:]
    w1v, w2v = w_vc[: STRIDE * DV], w_vc[STRIDE * DV:]

    cmp_k, cmp_v = pl.pallas_call(
        _compress_kernel,
        out_shape=(
            jax.ShapeDtypeStruct((KVH, NCP, D), jnp.float32),
            jax.ShapeDtypeStruct((KVH, NCP, DV), jnp.float32),
        ),
    )(kr, vr, w1k, w2k, w1v, w2v)

    qg = qs.reshape(S, KVH, G, D).transpose(1, 2, 0, 3)
    m = _overlap_matrix()
    ri, rj = _rank_matrices()
    cmp_o, selm = pl.pallas_call(
        _cmp_attn_kernel,
        grid=(KVH, S // QT),
        in_specs=[
            pl.BlockSpec((1, G, QT, D), lambda h, i: (h, 0, i, 0)),
            pl.BlockSpec((1, NCP, D), lambda h, i: (h, 0, 0)),
            pl.BlockSpec((1, NCP, DV), lambda h, i: (h, 0, 0)),
            pl.BlockSpec((NCP, NB), lambda h, i: (0, 0)),
            pl.BlockSpec((NB, NB * NB), lambda h, i: (0, 0)),
            pl.BlockSpec((NB, NB * NB), lambda h, i: (0, 0)),
        ],
        out_specs=(
            pl.BlockSpec((1, G, QT, DV), lambda h, i: (h, 0, i, 0)),
            pl.BlockSpec((1, QT, 128), lambda h, i: (h, i, 0)),
        ),
        out_shape=(
            jax.ShapeDtypeStruct((KVH, G, S, DV), jnp.float32),
            jax.ShapeDtypeStruct((KVH, S, 128), jnp.float32),
        ),
    )(qg, cmp_k, cmp_v, m, ri, rj)

    kt = ks.transpose(1, 0, 2)
    vt = vs.transpose(1, 0, 2)
    e = _expand_matrix()
    wg = jnp.zeros((8, D), jnp.float32).at[:3].set(w_gate)
    bg = jnp.zeros((1, 8), jnp.float32).at[0, :3].set(b_gate)

    outs = []
    for c in range(S // QTS):
        qbase = c * QTS
        kl = qbase + QTS
        wlo = max(0, qbase - WIN)
        need_sel = qbase >= S // 2
        outs.append(pl.pallas_call(
            functools.partial(_selwin_kernel, qbase=qbase, kl=kl, wlo=wlo,
                              need_sel=need_sel),
            grid=(KVH,),
            in_specs=[
                pl.BlockSpec((1, G, QTS, D), lambda h, cc=c: (h, 0, cc, 0)),
                pl.BlockSpec((1, kl, D), lambda h: (h, 0, 0)),
                pl.BlockSpec((1, kl, DV), lambda h: (h, 0, 0)),
                pl.BlockSpec((1, QTS, 128), lambda h, cc=c: (h, cc, 0)),
                pl.BlockSpec((1, G, QTS, DV), lambda h, cc=c: (h, 0, cc, 0)),
                pl.BlockSpec((NB, kl), lambda h: (0, 0)),
                pl.BlockSpec((8, D), lambda h: (0, 0)),
                pl.BlockSpec((1, 8), lambda h: (0, 0)),
            ],
            out_specs=pl.BlockSpec((1, G, QTS, DV), lambda h: (h, 0, 0, 0)),
            out_shape=jax.ShapeDtypeStruct((KVH, G, QTS, DV), jnp.float32),
        )(qg, kt[:, :kl], vt[:, :kl], selm, cmp_o, e[:, :kl], wg, bg))

    out = jnp.concatenate(outs, axis=2)
    return out.transpose(2, 0, 1, 3).reshape(1, S, QH, DV)


# speedup vs baseline: 1.1571x; 1.0084x over previous
import functools

import jax
import jax.numpy as jnp
import numpy as np
from jax.experimental import pallas as pl
from jax.experimental.pallas import tpu as pltpu

B, S, QH, KVH = 1, 2048, 12, 4
D, DV = 64, 64
KS, STRIDE, SEL, TOPN, WIN = 32, 16, 64, 16, 512
SCALE = D ** -0.5
NC = (S - KS) // STRIDE + 1
NCP = 128
NB = S // SEL
G = QH // KVH
NCHUNK = S // STRIDE
QT = 512
QTS = 512

_NEG_INF = float("-inf")


def _masked_softmax(s, mask):
    s = jnp.where(mask, s, _NEG_INF)
    m = jnp.max(s, axis=-1, keepdims=True)
    m = jnp.where(jnp.isfinite(m), m, 0.0)
    e = jnp.exp(s - m)
    return e / jnp.maximum(jnp.sum(e, axis=-1, keepdims=True), 1e-30)


def _cmp_attn_kernel(qt_ref, kr_ref, vr_ref, w1k_ref, w2k_ref, w1v_ref,
                     w2v_ref, m_ref, ri_ref, rj_ref,
                     cmp_o_ref, selm_ref, ck_s, cv_s):
    i = pl.program_id(1)
    base = i * QT

    @pl.when(i == 0)
    def _compress():
        ck0 = kr_ref[0]
        ckr = jnp.roll(ck0, -1, axis=0)
        ck_s[...] = (jnp.dot(ck0, w1k_ref[...], preferred_element_type=jnp.float32)
                     + jnp.dot(ckr, w2k_ref[...], preferred_element_type=jnp.float32))
        cv0 = vr_ref[0]
        cvr = jnp.roll(cv0, -1, axis=0)
        cv_s[...] = (jnp.dot(cv0, w1v_ref[...], preferred_element_type=jnp.float32)
                     + jnp.dot(cvr, w2v_ref[...], preferred_element_type=jnp.float32))

    qb = qt_ref[0].reshape(G * QT, D)
    ck = ck_s[...]
    cv = cv_s[...]
    s = jax.lax.dot_general(qb, ck, (((1,), (1,)), ((), ())),
                            preferred_element_type=jnp.float32) * SCALE
    row = jax.lax.broadcasted_iota(jnp.int32, (G * QT, NCP), 0)
    col = jax.lax.broadcasted_iota(jnp.int32, (G * QT, NCP), 1)
    t = base + row % QT
    mask = t >= (col * STRIDE + KS - 1)
    p = _masked_softmax(s, mask)
    cmp_o_ref[0] = jax.lax.dot_general(
        p, cv, (((1,), (0,)), ((), ())),
        preferred_element_type=jnp.float32).reshape(G, QT, DV)

    @pl.when(base >= S // 2)
    def _selmask():
        pj = jax.lax.dot_general(p, m_ref[...], (((1,), (0,)), ((), ())),
                                 preferred_element_type=jnp.float32)
        pj = pj.reshape(G, QT, NB).sum(axis=0)
        trow = base + jax.lax.broadcasted_iota(jnp.int32, (QT, NB), 0)
        jcol = jax.lax.broadcasted_iota(jnp.int32, (QT, NB), 1)
        cur = trow // SEL
        force = (jcol == cur).astype(jnp.float32) + (jcol == 0).astype(jnp.float32)
        sc = pj + 1e9 * force
        xi = jax.lax.dot_general(sc, ri_ref[...], (((1,), (0,)), ((), ())),
                                 preferred_element_type=jnp.float32)
        xj = jax.lax.dot_general(sc, rj_ref[...], (((1,), (0,)), ((), ())),
                                 preferred_element_type=jnp.float32)
        f = jax.lax.broadcasted_iota(jnp.int32, (QT, NB * NB), 1)
        ii = f % NB
        jj = f // NB
        beats = ((xi > xj) | ((xi == xj) & (ii < jj))).astype(jnp.float32)
        rank = jax.lax.dot_general(beats, rj_ref[...], (((1,), (1,)), ((), ())),
                                   preferred_element_type=jnp.float32)
        selm = (rank < TOPN).astype(jnp.float32)
        selm_ref[0] = jnp.concatenate(
            [selm, jnp.zeros((QT, 128 - NB), jnp.float32)], axis=1)


def _selwin_kernel(qt_ref, kt_ref, vt_ref, selm_ref, cmp_o_ref, e_ref, wg_ref,
                   bg_ref, out_ref, *, qbase, kl, wlo, need_sel):
    rows = G * QTS
    qb = qt_ref[0].reshape(rows, D)
    kh = kt_ref[0]
    vh = vt_ref[0]
    s = jax.lax.dot_general(qb, kh, (((1,), (1,)), ((), ())),
                            preferred_element_type=jnp.float32) * SCALE
    row = jax.lax.broadcasted_iota(jnp.int32, (rows, kl), 0)
    tk = jax.lax.broadcasted_iota(jnp.int32, (rows, kl), 1)
    tq = qbase + row % QTS
    causal = tq >= tk

    if need_sel:
        posf = jax.lax.dot_general(selm_ref[0][:, :NB], e_ref[...],
                                   (((1,), (0,)), ((), ())),
                                   preferred_element_type=jnp.float32)
        pos = jnp.concatenate([posf] * G, axis=0) > 0.5
        sel_m = pos & causal
    else:
        sel_m = causal
    p_sel = _masked_softmax(s, sel_m)
    o_sel = jax.lax.dot_general(p_sel, vh, (((1,), (0,)), ((), ())),
                                preferred_element_type=jnp.float32)

    single = (not need_sel) and wlo == 0 and qbase + QTS <= WIN + 1
    if single:
        o_win = o_sel
    else:
        sw = s[:, wlo:kl]
        win_m = causal[:, wlo:kl] & (tk[:, wlo:kl] >= tq[:, :1] - WIN)
        p_win = _masked_softmax(sw, win_m)
        o_win = jax.lax.dot_general(p_win, vh[wlo:kl], (((1,), (0,)), ((), ())),
                                    preferred_element_type=jnp.float32)

    z = jax.lax.dot_general(qb, wg_ref[...], (((1,), (1,)), ((), ())),
                            preferred_element_type=jnp.float32) + bg_ref[...]
    gate = jax.nn.sigmoid(z)
    cmp_rows = cmp_o_ref[0].reshape(rows, DV)
    out = (gate[:, 0:1] * cmp_rows + gate[:, 1:2] * o_sel + gate[:, 2:3] * o_win)
    out_ref[0] = out.reshape(G, QTS, DV)


def _expand_matrix():
    e = (np.arange(S)[None, :] // SEL == np.arange(NB)[:, None]).astype(np.float32)
    return jnp.asarray(e)


def _overlap_matrix():
    cmp_start = np.arange(NCP) * STRIDE
    sel_start = np.arange(NB) * SEL
    ov = ((cmp_start[:, None] < sel_start[None, :] + SEL)
          & (cmp_start[:, None] + KS > sel_start[None, :])).astype(np.float32)
    ov[NC:] = 0.0
    return jnp.asarray(ov)


def _rank_matrices():
    f = np.arange(NB * NB)
    ri = (f[None, :] % NB == np.arange(NB)[:, None]).astype(np.float32)
    rj = (f[None, :] // NB == np.arange(NB)[:, None]).astype(np.float32)
    return jnp.asarray(ri), jnp.asarray(rj)


@jax.jit
def kernel(q, k, v, w_kc, w_vc, w_gate, b_gate):
    qs = q[0]
    ks = k[0]
    vs = v[0]
    kr = ks.transpose(1, 0, 2).reshape(KVH, NCHUNK, STRIDE * D)
    vr = vs.transpose(1, 0, 2).reshape(KVH, NCHUNK, STRIDE * DV)
    w1k, w2k = w_kc[: STRIDE * D], w_kc[STRIDE * D:]
    w1v, w2v = w_vc[: STRIDE * DV], w_vc[STRIDE * DV:]

    qg = qs.reshape(S, KVH, G, D).transpose(1, 2, 0, 3)
    m = _overlap_matrix()
    ri, rj = _rank_matrices()
    cmp_o, selm = pl.pallas_call(
        _cmp_attn_kernel,
        grid=(KVH, S // QT),
        in_specs=[
            pl.BlockSpec((1, G, QT, D), lambda h, i: (h, 0, i, 0)),
            pl.BlockSpec((1, NCHUNK, STRIDE * D), lambda h, i: (h, 0, 0)),
            pl.BlockSpec((1, NCHUNK, STRIDE * DV), lambda h, i: (h, 0, 0)),
            pl.BlockSpec((STRIDE * D, D), lambda h, i: (0, 0)),
            pl.BlockSpec((STRIDE * D, D), lambda h, i: (0, 0)),
            pl.BlockSpec((STRIDE * DV, DV), lambda h, i: (0, 0)),
            pl.BlockSpec((STRIDE * DV, DV), lambda h, i: (0, 0)),
            pl.BlockSpec((NCP, NB), lambda h, i: (0, 0)),
            pl.BlockSpec((NB, NB * NB), lambda h, i: (0, 0)),
            pl.BlockSpec((NB, NB * NB), lambda h, i: (0, 0)),
        ],
        out_specs=(
            pl.BlockSpec((1, G, QT, DV), lambda h, i: (h, 0, i, 0)),
            pl.BlockSpec((1, QT, 128), lambda h, i: (h, i, 0)),
        ),
        out_shape=(
            jax.ShapeDtypeStruct((KVH, G, S, DV), jnp.float32),
            jax.ShapeDtypeStruct((KVH, S, 128), jnp.float32),
        ),
        scratch_shapes=[
            pltpu.VMEM((NCP, D), jnp.float32),
            pltpu.VMEM((NCP, DV), jnp.float32),
        ],
    )(qg, kr, vr, w1k, w2k, w1v, w2v, m, ri, rj)

    kt = ks.transpose(1, 0, 2)
    vt = vs.transpose(1, 0, 2)
    e = _expand_matrix()
    wg = jnp.zeros((8, D), jnp.float32).at[:3].set(w_gate)
    bg = jnp.zeros((1, 8), jnp.float32).at[0, :3].set(b_gate)

    outs = []
    for c in range(S // QTS):
        qbase = c * QTS
        kl = qbase + QTS
        wlo = max(0, qbase - WIN)
        need_sel = qbase >= S // 2
        outs.append(pl.pallas_call(
            functools.partial(_selwin_kernel, qbase=qbase, kl=kl, wlo=wlo,
                              need_sel=need_sel),
            grid=(KVH,),
            in_specs=[
                pl.BlockSpec((1, G, QTS, D), lambda h, cc=c: (h, 0, cc, 0)),
                pl.BlockSpec((1, kl, D), lambda h: (h, 0, 0)),
                pl.BlockSpec((1, kl, DV), lambda h: (h, 0, 0)),
                pl.BlockSpec((1, QTS, 128), lambda h, cc=c: (h, cc, 0)),
                pl.BlockSpec((1, G, QTS, DV), lambda h, cc=c: (h, 0, cc, 0)),
                pl.BlockSpec((NB, kl), lambda h: (0, 0)),
                pl.BlockSpec((8, D), lambda h: (0, 0)),
                pl.BlockSpec((1, 8), lambda h: (0, 0)),
            ],
            out_specs=pl.BlockSpec((1, G, QTS, DV), lambda h: (h, 0, 0, 0)),
            out_shape=jax.ShapeDtypeStruct((KVH, G, QTS, DV), jnp.float32),
        )(qg, kt[:, :kl], vt[:, :kl], selm, cmp_o, e[:, :kl], wg, bg))

    out = jnp.concatenate(outs, axis=2)
    return out.transpose(2, 0, 1, 3).reshape(1, S, QH, DV)


# speedup vs baseline: 1.1934x; 1.0314x over previous
import functools

import jax
import jax.numpy as jnp
import numpy as np
from jax.experimental import pallas as pl
from jax.experimental.pallas import tpu as pltpu

B, S, QH, KVH = 1, 2048, 12, 4
D, DV = 64, 64
KS, STRIDE, SEL, TOPN, WIN = 32, 16, 64, 16, 512
SCALE = D ** -0.5
NC = (S - KS) // STRIDE + 1
NCP = 128
NB = S // SEL
G = QH // KVH
NCHUNK = S // STRIDE
QT = 1024
QTS = 512

_NEG_INF = float("-inf")


def _masked_softmax(s, mask):
    s = jnp.where(mask, s, _NEG_INF)
    m = jnp.max(s, axis=-1, keepdims=True)
    m = jnp.where(jnp.isfinite(m), m, 0.0)
    e = jnp.exp(s - m)
    return e / jnp.maximum(jnp.sum(e, axis=-1, keepdims=True), 1e-30)


def _cmp_attn_kernel(qt_ref, kr_ref, vr_ref, w1k_ref, w2k_ref, w1v_ref,
                     w2v_ref, m_ref, ri_ref, rj_ref,
                     cmp_o_ref, selm_ref, ck_s, cv_s):
    i = pl.program_id(1)
    base = i * QT

    @pl.when(i == 0)
    def _compress():
        ck0 = kr_ref[0]
        ckr = jnp.roll(ck0, -1, axis=0)
        ck_s[...] = (jnp.dot(ck0, w1k_ref[...], preferred_element_type=jnp.float32)
                     + jnp.dot(ckr, w2k_ref[...], preferred_element_type=jnp.float32))
        cv0 = vr_ref[0]
        cvr = jnp.roll(cv0, -1, axis=0)
        cv_s[...] = (jnp.dot(cv0, w1v_ref[...], preferred_element_type=jnp.float32)
                     + jnp.dot(cvr, w2v_ref[...], preferred_element_type=jnp.float32))

    qb = qt_ref[0].reshape(G * QT, D)
    ck = ck_s[...]
    cv = cv_s[...]
    s = jax.lax.dot_general(qb, ck, (((1,), (1,)), ((), ())),
                            preferred_element_type=jnp.float32) * SCALE
    row = jax.lax.broadcasted_iota(jnp.int32, (G * QT, NCP), 0)
    col = jax.lax.broadcasted_iota(jnp.int32, (G * QT, NCP), 1)
    t = base + row % QT
    mask = t >= (col * STRIDE + KS - 1)
    p = _masked_softmax(s, mask)
    cmp_o_ref[0] = jax.lax.dot_general(
        p, cv, (((1,), (0,)), ((), ())),
        preferred_element_type=jnp.float32).reshape(G, QT, DV)

    @pl.when(base >= S // 2)
    def _selmask():
        pj = jax.lax.dot_general(p, m_ref[...], (((1,), (0,)), ((), ())),
                                 preferred_element_type=jnp.float32)
        pj = pj.reshape(G, QT, NB).sum(axis=0)
        trow = base + jax.lax.broadcasted_iota(jnp.int32, (QT, NB), 0)
        jcol = jax.lax.broadcasted_iota(jnp.int32, (QT, NB), 1)
        cur = trow // SEL
        force = (jcol == cur).astype(jnp.float32) + (jcol == 0).astype(jnp.float32)
        sc = pj + 1e9 * force
        xi = jax.lax.dot_general(sc, ri_ref[...], (((1,), (0,)), ((), ())),
                                 preferred_element_type=jnp.float32)
        xj = jax.lax.dot_general(sc, rj_ref[...], (((1,), (0,)), ((), ())),
                                 preferred_element_type=jnp.float32)
        f = jax.lax.broadcasted_iota(jnp.int32, (QT, NB * NB), 1)
        ii = f % NB
        jj = f // NB
        beats = ((xi > xj) | ((xi == xj) & (ii < jj))).astype(jnp.float32)
        rank = jax.lax.dot_general(beats, rj_ref[...], (((1,), (1,)), ((), ())),
                                   preferred_element_type=jnp.float32)
        selm = (rank < TOPN).astype(jnp.float32)
        selm_ref[0] = jnp.concatenate(
            [selm, jnp.zeros((QT, 128 - NB), jnp.float32)], axis=1)


def _selwin_kernel(qt_ref, kt_ref, vt_ref, selm_ref, cmp_o_ref, e_ref, wg_ref,
                   bg_ref, out_ref, *, qbase, kl, wlo, need_sel):
    rows = G * QTS
    qb = qt_ref[0].reshape(rows, D)
    kh = kt_ref[0]
    vh = vt_ref[0]
    s = jax.lax.dot_general(qb, kh, (((1,), (1,)), ((), ())),
                            preferred_element_type=jnp.float32) * SCALE
    row = jax.lax.broadcasted_iota(jnp.int32, (rows, kl), 0)
    tk = jax.lax.broadcasted_iota(jnp.int32, (rows, kl), 1)
    tq = qbase + row % QTS
    causal = tq >= tk

    if need_sel:
        posf = jax.lax.dot_general(selm_ref[0][:, :NB], e_ref[...],
                                   (((1,), (0,)), ((), ())),
                                   preferred_element_type=jnp.float32)
        pos = jnp.concatenate([posf] * G, axis=0) > 0.5
        sel_m = pos & causal
    else:
        sel_m = causal
    p_sel = _masked_softmax(s, sel_m)
    o_sel = jax.lax.dot_general(p_sel, vh, (((1,), (0,)), ((), ())),
                                preferred_element_type=jnp.float32)

    single = (not need_sel) and wlo == 0 and qbase + QTS <= WIN + 1
    if single:
        o_win = o_sel
    else:
        sw = s[:, wlo:kl]
        win_m = causal[:, wlo:kl] & (tk[:, wlo:kl] >= tq[:, :1] - WIN)
        p_win = _masked_softmax(sw, win_m)
        o_win = jax.lax.dot_general(p_win, vh[wlo:kl], (((1,), (0,)), ((), ())),
                                    preferred_element_type=jnp.float32)

    z = jax.lax.dot_general(qb, wg_ref[...], (((1,), (1,)), ((), ())),
                            preferred_element_type=jnp.float32) + bg_ref[...]
    gate = jax.nn.sigmoid(z)
    cmp_rows = cmp_o_ref[0].reshape(rows, DV)
    out = (gate[:, 0:1] * cmp_rows + gate[:, 1:2] * o_sel + gate[:, 2:3] * o_win)
    out_ref[0] = out.reshape(G, QTS, DV)


def _expand_matrix():
    e = (np.arange(S)[None, :] // SEL == np.arange(NB)[:, None]).astype(np.float32)
    return jnp.asarray(e)


def _overlap_matrix():
    cmp_start = np.arange(NCP) * STRIDE
    sel_start = np.arange(NB) * SEL
    ov = ((cmp_start[:, None] < sel_start[None, :] + SEL)
          & (cmp_start[:, None] + KS > sel_start[None, :])).astype(np.float32)
    ov[NC:] = 0.0
    return jnp.asarray(ov)


def _rank_matrices():
    f = np.arange(NB * NB)
    ri = (f[None, :] % NB == np.arange(NB)[:, None]).astype(np.float32)
    rj = (f[None, :] // NB == np.arange(NB)[:, None]).astype(np.float32)
    return jnp.asarray(ri), jnp.asarray(rj)


@jax.jit
def kernel(q, k, v, w_kc, w_vc, w_gate, b_gate):
    qs = q[0]
    ks = k[0]
    vs = v[0]
    kr = ks.transpose(1, 0, 2).reshape(KVH, NCHUNK, STRIDE * D)
    vr = vs.transpose(1, 0, 2).reshape(KVH, NCHUNK, STRIDE * DV)
    w1k, w2k = w_kc[: STRIDE * D], w_kc[STRIDE * D:]
    w1v, w2v = w_vc[: STRIDE * DV], w_vc[STRIDE * DV:]

    qg = qs.reshape(S, KVH, G, D).transpose(1, 2, 0, 3)
    m = _overlap_matrix()
    ri, rj = _rank_matrices()
    cmp_o, selm = pl.pallas_call(
        _cmp_attn_kernel,
        grid=(KVH, S // QT),
        in_specs=[
            pl.BlockSpec((1, G, QT, D), lambda h, i: (h, 0, i, 0)),
            pl.BlockSpec((1, NCHUNK, STRIDE * D), lambda h, i: (h, 0, 0)),
            pl.BlockSpec((1, NCHUNK, STRIDE * DV), lambda h, i: (h, 0, 0)),
            pl.BlockSpec((STRIDE * D, D), lambda h, i: (0, 0)),
            pl.BlockSpec((STRIDE * D, D), lambda h, i: (0, 0)),
            pl.BlockSpec((STRIDE * DV, DV), lambda h, i: (0, 0)),
            pl.BlockSpec((STRIDE * DV, DV), lambda h, i: (0, 0)),
            pl.BlockSpec((NCP, NB), lambda h, i: (0, 0)),
            pl.BlockSpec((NB, NB * NB), lambda h, i: (0, 0)),
            pl.BlockSpec((NB, NB * NB), lambda h, i: (0, 0)),
        ],
        out_specs=(
            pl.BlockSpec((1, G, QT, DV), lambda h, i: (h, 0, i, 0)),
            pl.BlockSpec((1, QT, 128), lambda h, i: (h, i, 0)),
        ),
        out_shape=(
            jax.ShapeDtypeStruct((KVH, G, S, DV), jnp.float32),
            jax.ShapeDtypeStruct((KVH, S, 128), jnp.float32),
        ),
        scratch_shapes=[
            pltpu.VMEM((NCP, D), jnp.float32),
            pltpu.VMEM((NCP, DV), jnp.float32),
        ],
    )(qg, kr, vr, w1k, w2k, w1v, w2v, m, ri, rj)

    kt = ks.transpose(1, 0, 2)
    vt = vs.transpose(1, 0, 2)
    e = _expand_matrix()
    wg = jnp.zeros((8, D), jnp.float32).at[:3].set(w_gate)
    bg = jnp.zeros((1, 8), jnp.float32).at[0, :3].set(b_gate)

    outs = []
    for c in range(S // QTS):
        qbase = c * QTS
        kl = qbase + QTS
        wlo = max(0, qbase - WIN)
        need_sel = qbase >= S // 2
        outs.append(pl.pallas_call(
            functools.partial(_selwin_kernel, qbase=qbase, kl=kl, wlo=wlo,
                              need_sel=need_sel),
            grid=(KVH,),
            in_specs=[
                pl.BlockSpec((1, G, QTS, D), lambda h, cc=c: (h, 0, cc, 0)),
                pl.BlockSpec((1, kl, D), lambda h: (h, 0, 0)),
                pl.BlockSpec((1, kl, DV), lambda h: (h, 0, 0)),
                pl.BlockSpec((1, QTS, 128), lambda h, cc=c: (h, cc, 0)),
                pl.BlockSpec((1, G, QTS, DV), lambda h, cc=c: (h, 0, cc, 0)),
                pl.BlockSpec((NB, kl), lambda h: (0, 0)),
                pl.BlockSpec((8, D), lambda h: (0, 0)),
                pl.BlockSpec((1, 8), lambda h: (0, 0)),
            ],
            out_specs=pl.BlockSpec((1, G, QTS, DV), lambda h: (h, 0, 0, 0)),
            out_shape=jax.ShapeDtypeStruct((KVH, G, QTS, DV), jnp.float32),
        )(qg, kt[:, :kl], vt[:, :kl], selm, cmp_o, e[:, :kl], wg, bg))

    out = jnp.concatenate(outs, axis=2)
    return out.transpose(2, 0, 1, 3).reshape(1, S, QH, DV)
